# baseline (speedup 1.0000x reference)
import jax
import jax.numpy as jnp
from jax.experimental import pallas as pl
from jax.experimental.pallas import tpu as pltpu

_BI = 256
_BCC = 256
_CSTART = (4, 5, 9, 10)


def _fused_kernel(x_ref, p_ref, c_hbm, xhv_ref, preds_ref,
                  xb_s, pbf_s, g_s, fac_s, cbuf_s, csem):
    i = pl.program_id(0)
    half = pl.num_programs(0) // 2

    @pl.when(i == 0)
    def _prep():
        pb = p_ref[...].astype(jnp.bfloat16)
        pbf_s[...] = pb
        g_s[...] = jax.lax.dot_general(
            pb, pb, (((0,), (0,)), ((), ())),
            preferred_element_type=jnp.float32).astype(jnp.bfloat16)

    @pl.when(i < half)
    def _ingest_step():
        xf = x_ref[...]
        xb = xf.astype(jnp.bfloat16)
        xb_s[pl.ds(i * 2 * _BI, 2 * _BI), :] = xb
        t = jax.lax.dot_general(
            xb, g_s[...], (((1,), (1,)), ((), ())),
            preferred_element_type=jnp.float32)
        s = jnp.sum(t * xf, axis=1, keepdims=True)
        fac_s[pl.ds(i * 2 * _BI, 2 * _BI), :] = 1.0 / (jnp.sqrt(s) + 1e-12)

    xb_i = xb_s[pl.ds(i * _BI, _BI), :]
    xhv_ref[...] = jax.lax.dot_general(
        xb_i, pbf_s[...], (((1,), (1,)), ((), ())),
        preferred_element_type=jnp.float32)

    for j in range(4):
        slot = j % 2

        @pl.when(i == half + j)
        def _centroid_step(j=j, slot=slot):
            pltpu.make_async_copy(
                c_hbm.at[pl.ds(j * _BCC, _BCC), :],
                cbuf_s.at[slot],
                csem.at[slot],
            ).wait()
            c = cbuf_s[slot]
            cs = jnp.sum(c * c, axis=1, keepdims=True)
            cn = (c * (1.0 / (jnp.sqrt(cs) + 1e-12))).astype(jnp.bfloat16)
            mb = jax.lax.dot_general(
                cn, pbf_s[...], (((1,), (0,)), ((), ())),
                preferred_element_type=jnp.float32).astype(jnp.bfloat16)
            pc = jax.lax.dot_general(
                xb_s[...], mb, (((1,), (1,)), ((), ())),
                preferred_element_type=jnp.float32)
            preds_ref[...] = pc * fac_s[...]

        @pl.when(i == _CSTART[j])
        def _start_copy(j=j, slot=slot):
            pltpu.make_async_copy(
                c_hbm.at[pl.ds(j * _BCC, _BCC), :],
                cbuf_s.at[slot],
                csem.at[slot],
            ).start()


@jax.jit
def kernel(x, projection, centroids):
    B, F = x.shape
    D, _ = projection.shape
    C, _ = centroids.shape
    NS = B // _BI

    xhv, preds = pl.pallas_call(
        _fused_kernel,
        grid=(NS,),
        in_specs=[
            pl.BlockSpec((2 * _BI, F), lambda i: (jnp.minimum(i, 7), 0)),
            pl.BlockSpec((D, F), lambda i: (0, 0)),
            pl.BlockSpec(memory_space=pltpu.MemorySpace.HBM),
        ],
        out_specs=[
            pl.BlockSpec((_BI, D), lambda i: (i, 0)),
            pl.BlockSpec((B, _BCC),
                         lambda i: (0, jnp.clip(i - 8, 0, 3))),
        ],
        out_shape=[
            jax.ShapeDtypeStruct((B, D), jnp.float32),
            jax.ShapeDtypeStruct((B, C), jnp.float32),
        ],
        scratch_shapes=[
            pltpu.VMEM((B, F), jnp.bfloat16),
            pltpu.VMEM((D, F), jnp.bfloat16),
            pltpu.VMEM((F, F), jnp.bfloat16),
            pltpu.VMEM((B, 1), jnp.float32),
            pltpu.VMEM((2, _BCC, D), jnp.float32),
            pltpu.SemaphoreType.DMA((2,)),
        ],
        compiler_params=pltpu.CompilerParams(
            dimension_semantics=("arbitrary",),
            vmem_limit_bytes=100 * 1024 * 1024,
        ),
    )(x, projection, centroids)
    return (preds, xhv)


# speedup vs baseline: 1.1808x; 1.1808x over previous
import jax
import jax.numpy as jnp
from jax.experimental import pallas as pl
from jax.experimental.pallas import tpu as pltpu

_BI = 256
_BCC = 256


def _fused_kernel(x_ref, p_ref, c_ref, xhv_ref, preds_ref,
                  xb_s, pbf_s, g_s, fac_s):
    i = pl.program_id(0)
    half = pl.num_programs(0) // 2

    @pl.when(i == 0)
    def _prep():
        pb = p_ref[...].astype(jnp.bfloat16)
        pbf_s[...] = pb
        g_s[...] = jax.lax.dot_general(
            pb, pb, (((0,), (0,)), ((), ())),
            preferred_element_type=jnp.float32).astype(jnp.bfloat16)

    @pl.when(i < half)
    def _ingest_step():
        xf = x_ref[...]
        xb = xf.astype(jnp.bfloat16)
        xb_s[pl.ds(i * 2 * _BI, 2 * _BI), :] = xb
        t = jax.lax.dot_general(
            xb, g_s[...], (((1,), (1,)), ((), ())),
            preferred_element_type=jnp.float32)
        s = jnp.sum(t * xf, axis=1, keepdims=True)
        fac_s[pl.ds(i * 2 * _BI, 2 * _BI), :] = 1.0 / (jnp.sqrt(s) + 1e-12)

    xb_i = xb_s[pl.ds(i * _BI, _BI), :]
    xhv_ref[...] = jax.lax.dot_general(
        xb_i, pbf_s[...], (((1,), (1,)), ((), ())),
        preferred_element_type=jnp.float32)

    @pl.when((i >= half) & (i < half + 4))
    def _centroid_step():
        c = c_ref[...]
        cs = jnp.sum(c * c, axis=1, keepdims=True)
        cn = (c * (1.0 / (jnp.sqrt(cs) + 1e-12))).astype(jnp.bfloat16)
        mb = jax.lax.dot_general(
            cn, pbf_s[...], (((1,), (0,)), ((), ())),
            preferred_element_type=jnp.float32).astype(jnp.bfloat16)
        pc = jax.lax.dot_general(
            xb_s[...], mb, (((1,), (1,)), ((), ())),
            preferred_element_type=jnp.float32)
        preds_ref[...] = pc * fac_s[...]


@jax.jit
def kernel(x, projection, centroids):
    B, F = x.shape
    D, _ = projection.shape
    C, _ = centroids.shape
    NS = B // _BI

    xhv, preds = pl.pallas_call(
        _fused_kernel,
        grid=(NS,),
        in_specs=[
            pl.BlockSpec((2 * _BI, F), lambda i: (jnp.minimum(i, 7), 0)),
            pl.BlockSpec((D, F), lambda i: (0, 0)),
            pl.BlockSpec((_BCC, D),
                         lambda i: (jnp.clip(i - 8, 0, 3), 0)),
        ],
        out_specs=[
            pl.BlockSpec((_BI, D), lambda i: (i, 0)),
            pl.BlockSpec((B, _BCC),
                         lambda i: (0, jnp.clip(i - 8, 0, 3))),
        ],
        out_shape=[
            jax.ShapeDtypeStruct((B, D), jnp.float32),
            jax.ShapeDtypeStruct((B, C), jnp.float32),
        ],
        scratch_shapes=[
            pltpu.VMEM((B, F), jnp.bfloat16),
            pltpu.VMEM((D, F), jnp.bfloat16),
            pltpu.VMEM((F, F), jnp.bfloat16),
            pltpu.VMEM((B, 1), jnp.float32),
        ],
        compiler_params=pltpu.CompilerParams(
            dimension_semantics=("arbitrary",),
            vmem_limit_bytes=64 * 1024 * 1024,
        ),
    )(x, projection, centroids)
    return (preds, xhv)
